# baseline (speedup 1.0000x reference)
import functools

import jax
import jax.numpy as jnp
from jax import lax
from jax.experimental import pallas as pl
from jax.experimental.pallas import tpu as pltpu
from jax.experimental.pallas import tpu_sc as plsc

NU = 50000
NI = 50000
E = 800000
DIN = 128
H = 64
DOUT = 32

NP = 50048
DUMP = 50000
NTILE = 16
TS = NP // NTILE

EP1 = 835584
ER1 = EP1 // 128
ROWS1 = ER1 // NTILE
NB1 = 2
NCH1 = ROWS1 // NB1

EP2 = 819200
ER2 = EP2 // 128
ROWS2 = ER2 // NTILE
NB2 = 8
NCH2 = ROWS2 // NB2
ROWS_C = ER2 // 2 // NTILE
NBC = 10
NCHC = ROWS_C // NBC

BLK = 2000


def _sc_segsum3(w2, nb, nbuf, er, rows_t):
  mesh = plsc.VectorSubcoreMesh(core_axis_name="c", subcore_axis_name="s")
  out1 = jax.ShapeDtypeStruct((2 * NP, w2), jnp.float32)
  nch = rows_t // nb
  assert nch % nbuf == 0
  buf = lambda: [pltpu.VMEM((nb, 128), jnp.int32),
                 pltpu.VMEM((nb, 128), jnp.int32),
                 pltpu.VMEM((nb, 128, w2), jnp.float32),
                 pltpu.SemaphoreType.DMA,
                 pltpu.SemaphoreType.DMA]
  scratch = []
  for _ in range(nbuf):
    scratch += buf()

  @functools.partial(
      pl.kernel,
      out_type=(out1, out1, out1),
      mesh=mesh,
      compiler_params=pltpu.CompilerParams(use_tc_tiling_on_sc=False),
      scratch_types=scratch + [pltpu.VMEM_SHARED((NP, w2), jnp.float32)],
  )
  def k(*refs):
    (ya, srca, dsta, yb, srcb, dstb, yc, srcc, dstc, zeros_hbm,
     outa, outb, outc) = refs[:13]
    acc = refs[-1]
    bufs = tuple(tuple(refs[13 + 5 * q:13 + 5 * (q + 1)])
                 for q in range(nbuf))
    c = lax.axis_index("c")
    s = lax.axis_index("s")

    def drain_sc(rows, ssem):
      for j in range(nb):
        pltpu.make_async_copy(rows.at[j], acc.at[pl.ds(0, 128)], ssem).wait()

    for y, src2, dst2, out in ((ya, srca, dsta, outa),
                               (yb, srcb, dstb, outb),
                               (yc, srcc, dstc, outc)):
      pltpu.sync_copy(zeros_hbm.at[pl.ds(0, TS)], acc.at[pl.ds(s * TS, TS)])
      plsc.subcore_barrier()
      src_base = c * er + s * rows_t
      dst_base = s * rows_t

      def ring(i2, carry):
        gcps = []
        for b, (srcv, dstv, rows, gsem, ssem) in enumerate(bufs):
          ci = nbuf * i2 + b

          @pl.when(i2 > 0)
          def _():
            drain_sc(rows, ssem)

          pltpu.sync_copy(src2.at[pl.ds(src_base + ci * nb, nb)], srcv)
          pltpu.sync_copy(dst2.at[pl.ds(dst_base + ci * nb, nb)], dstv)
          gcps.append([pltpu.async_copy(y.at[srcv.at[j]], rows.at[j], gsem)
                       for j in range(nb)])
        for b, (srcv, dstv, rows, gsem, ssem) in enumerate(bufs):
          for cp in gcps[b]:
            cp.wait()
          for j in range(nb):
            pltpu.async_copy(rows.at[j], acc.at[dstv.at[j]], ssem, add=True)
        return carry

      lax.fori_loop(0, nch // nbuf, ring, 0)
      for (srcv, dstv, rows, gsem, ssem) in bufs:
        drain_sc(rows, ssem)
      plsc.subcore_barrier()
      pltpu.sync_copy(acc.at[pl.ds(s * TS, TS)],
                      out.at[pl.ds(c * NP + s * TS, TS)])
    return

  return k


def _sc_counts():
  mesh = plsc.VectorSubcoreMesh(core_axis_name="c", subcore_axis_name="s")
  out1 = jax.ShapeDtypeStruct((2 * NP, 16), jnp.float32)

  @functools.partial(
      pl.kernel,
      out_type=(out1, out1, out1),
      mesh=mesh,
      compiler_params=pltpu.CompilerParams(use_tc_tiling_on_sc=False),
      scratch_types=[
          pltpu.VMEM((NBC, 128), jnp.int32),
          pltpu.VMEM((NBC, 128), jnp.int32),
          pltpu.VMEM((128, 16), jnp.float32),
          pltpu.VMEM_SHARED((NP, 16), jnp.float32),
          pltpu.SemaphoreType.DMA,
          pltpu.SemaphoreType.DMA,
      ],
  )
  def k(dsta, dstb, dstc, ones_hbm, zeros_hbm,
        outa, outb, outc, dstv0, dstv1, ones, acc, sem0, sem1):
    c = lax.axis_index("c")
    s = lax.axis_index("s")
    pltpu.sync_copy(ones_hbm, ones)
    bufs = ((dstv0, sem0), (dstv1, sem1))

    def drain(sem):
      for j in range(NBC):
        pltpu.make_async_copy(ones, acc.at[pl.ds(0, 128)], sem).wait()

    for dst2, out in ((dsta, outa), (dstb, outb), (dstc, outc)):
      pltpu.sync_copy(zeros_hbm.at[pl.ds(0, TS)], acc.at[pl.ds(s * TS, TS)])
      plsc.subcore_barrier()
      base = c * (ER2 // 2) + s * ROWS_C

      def pair(i2, carry):
        for b, (dstv, sem) in enumerate(bufs):
          @pl.when(i2 > 0)
          def _():
            drain(sem)

          pltpu.sync_copy(dst2.at[pl.ds(base + (2 * i2 + b) * NBC, NBC)], dstv)
          for j in range(NBC):
            pltpu.async_copy(ones, acc.at[dstv.at[j]], sem, add=True)
        return carry

      lax.fori_loop(0, NCHC // 2, pair, 0)
      for dstv, sem in bufs:
        drain(sem)
      plsc.subcore_barrier()
      pltpu.sync_copy(acc.at[pl.ds(s * TS, TS)],
                      out.at[pl.ds(c * NP + s * TS, TS)])
    return

  return k


_PREC = lax.Precision.DEFAULT


def _full(shape):
  return pl.BlockSpec(shape, lambda i: (0,) * len(shape))


def _dot(a, b):
  return jnp.dot(a, b, preferred_element_type=jnp.float32, precision=_PREC)


def _tc_pre(n, ny):
  grid = n // BLK
  in_specs = [pl.BlockSpec((BLK, DIN), lambda i: (i, 0)),
              _full((DIN, H)), _full((1, H))]
  in_specs += [_full((H, H))] * ny
  in_specs += [_full((H, H)), _full((1, H))]
  out_shape = tuple([jax.ShapeDtypeStruct((2, n, 32), jnp.float32)] * ny
                    + [jax.ShapeDtypeStruct((n, H), jnp.float32)])
  out_specs = tuple([pl.BlockSpec((2, BLK, 32), lambda i: (0, i, 0))] * ny
                    + [pl.BlockSpec((BLK, H), lambda i: (i, 0))])

  def body(*refs):
    x, pwt, pb = refs[0], refs[1], refs[2]
    wls = refs[3:3 + ny]
    wrt, blc = refs[3 + ny], refs[4 + ny]
    youts = refs[5 + ny:5 + 2 * ny]
    duo = refs[5 + 2 * ny]
    h = _dot(x[...], pwt[...]) + pb[...]
    for wl, yo in zip(wls, youts):
      yv = _dot(h, wl[...])
      yo[0] = yv[:, :32]
      yo[1] = yv[:, 32:]
    duo[...] = _dot(h, wrt[...]) + blc[...]

  return pl.pallas_call(body, grid=(grid,), in_specs=in_specs,
                        out_specs=out_specs, out_shape=out_shape)


def _seg_spec(w2):
  return pl.BlockSpec((2, BLK, w2), lambda i: (0, i, 0))


def _agg(p_ref, c_ref):
  inv = 1.0 / jnp.maximum(c_ref[0, :, 0:1] + c_ref[1, :, 0:1], 1.0)
  return jnp.concatenate([p_ref[0], p_ref[1]], axis=1) * inv


def _tc_mid(n, nrel, ny):
  grid = n // BLK
  in_specs = []
  for _ in range(nrel):
    in_specs += [_seg_spec(32), _seg_spec(16)]
  in_specs += [pl.BlockSpec((BLK, H), lambda i: (i, 0))]
  in_specs += [_full((H, DOUT))] * ny
  in_specs += [_full((H, DOUT)), _full((1, DOUT))]
  out_shape = tuple([jax.ShapeDtypeStruct((2, n, 16), jnp.float32)] * ny
                    + [jax.ShapeDtypeStruct((n, DOUT), jnp.float32)])
  out_specs = tuple([pl.BlockSpec((2, BLK, 16), lambda i: (0, i, 0))] * ny
                    + [pl.BlockSpec((BLK, DOUT), lambda i: (i, 0))])
  scale = 1.0 / nrel

  def body(*refs):
    pre = None
    for r in range(nrel):
      a = _agg(refs[2 * r], refs[2 * r + 1])
      pre = a if pre is None else pre + a
    d = refs[2 * nrel]
    wls = refs[2 * nrel + 1:2 * nrel + 1 + ny]
    wrt, blc = refs[2 * nrel + 1 + ny], refs[2 * nrel + 2 + ny]
    youts = refs[2 * nrel + 3 + ny:2 * nrel + 3 + 2 * ny]
    duo = refs[2 * nrel + 3 + 2 * ny]
    h1 = jnp.maximum((pre + d[...]) * scale, 0.0)
    for wl, yo in zip(wls, youts):
      yv = _dot(h1, wl[...])
      yo[0] = yv[:, :16]
      yo[1] = yv[:, 16:]
    duo[...] = _dot(h1, wrt[...]) + blc[...]

  return pl.pallas_call(body, grid=(grid,), in_specs=in_specs,
                        out_specs=out_specs, out_shape=out_shape)


def _tc_post(n, nrel):
  grid = n // BLK
  in_specs = []
  for _ in range(nrel):
    in_specs += [_seg_spec(16), _seg_spec(16)]
  in_specs += [pl.BlockSpec((BLK, DOUT), lambda i: (i, 0))]
  out_shape = jax.ShapeDtypeStruct((n, DOUT), jnp.float32)
  out_specs = pl.BlockSpec((BLK, DOUT), lambda i: (i, 0))
  scale = 1.0 / nrel

  def body(*refs):
    pre = None
    for r in range(nrel):
      a = _agg(refs[2 * r], refs[2 * r + 1])
      pre = a if pre is None else pre + a
    d = refs[2 * nrel]
    refs[2 * nrel + 1][...] = jnp.maximum((pre + d[...]) * scale, 0.0)

  return pl.pallas_call(body, grid=(grid,), in_specs=in_specs,
                        out_specs=out_specs, out_shape=out_shape)


def _prep_edges(ei, n_src):
  src = ei[0].astype(jnp.int32)
  dst = ei[1].astype(jnp.int32)
  p1 = EP1 - E
  p2 = EP2 - E
  src1 = jnp.concatenate([src, jnp.zeros((p1,), jnp.int32)])
  dst1 = jnp.concatenate([dst, jnp.full((p1,), DUMP, jnp.int32)])
  src2 = jnp.concatenate([src, jnp.zeros((p2,), jnp.int32)])
  dst2 = jnp.concatenate([dst, jnp.full((p2,), DUMP, jnp.int32)])
  sa = jnp.concatenate([src1, src1 + n_src]).reshape(2 * ER1, 128)
  sb = jnp.concatenate([src2, src2 + n_src]).reshape(2 * ER2, 128)
  return sa, dst1.reshape(ER1, 128), sb, dst2.reshape(ER2, 128)


def kernel(x_user, x_item, edge_index_social, edge_index_interacts,
           edge_index_rev_interacts, up_W, up_b, ip_W, ip_b,
           c1s_Wl, c1s_bl, c1s_Wr, c1i_Wl, c1i_bl, c1i_Wr,
           c1r_Wl, c1r_bl, c1r_Wr,
           c2s_Wl, c2s_bl, c2s_Wr, c2i_Wl, c2i_bl, c2i_Wr,
           c2r_Wl, c2r_bl, c2r_Wr):
  sa_s, da_s, sb_s, db_s = _prep_edges(edge_index_social, NU)
  sa_i, da_i, sb_i, db_i = _prep_edges(edge_index_interacts, NU)
  sa_r, da_r, sb_r, db_r = _prep_edges(edge_index_rev_interacts, NI)

  zeros32 = jnp.zeros((TS, 32), jnp.float32)
  zeros16 = jnp.zeros((TS, 16), jnp.float32)
  ones128 = jnp.ones((128, 16), jnp.float32)

  ys1, yi1, du1 = _tc_pre(NU, 2)(
      x_user, up_W.T, up_b.reshape(1, H),
      c1s_Wl.T, c1i_Wl.T,
      (c1s_Wr + c1r_Wr).T, (c1s_bl + c1r_bl).reshape(1, H))
  yr1, di1 = _tc_pre(NI, 1)(
      x_item, ip_W.T, ip_b.reshape(1, H),
      c1r_Wl.T,
      c1i_Wr.T, c1i_bl.reshape(1, H))

  cnt_s, cnt_i, cnt_r = _sc_counts()(db_s, db_i, db_r, ones128, zeros16)
  seg_s, seg_i, seg_r = _sc_segsum3(32, NB1, 3, ER1, ROWS1)(
      ys1.reshape(2 * NU, 32), sa_s, da_s,
      yi1.reshape(2 * NU, 32), sa_i, da_i,
      yr1.reshape(2 * NI, 32), sa_r, da_r,
      zeros32)

  r2 = lambda a, w: a.reshape(2, NP, w)
  ys2, yi2, du2 = _tc_mid(NU, 2, 2)(
      r2(seg_s, 32), r2(cnt_s, 16),
      r2(seg_r, 32), r2(cnt_r, 16),
      du1,
      c2s_Wl.T, c2i_Wl.T,
      (c2s_Wr + c2r_Wr).T, (c2s_bl + c2r_bl).reshape(1, DOUT))
  yr2, di2 = _tc_mid(NI, 1, 1)(
      r2(seg_i, 32), r2(cnt_i, 16),
      di1,
      c2r_Wl.T,
      c2i_Wr.T, c2i_bl.reshape(1, DOUT))

  s2_s, s2_i, s2_r = _sc_segsum3(16, NB2, 2, ER2, ROWS2)(
      ys2.reshape(2 * NU, 16), sb_s, db_s,
      yi2.reshape(2 * NU, 16), sb_i, db_i,
      yr2.reshape(2 * NI, 16), sb_r, db_r,
      zeros16)

  h2u = _tc_post(NU, 2)(
      r2(s2_s, 16), r2(cnt_s, 16),
      r2(s2_r, 16), r2(cnt_r, 16),
      du2)
  h2i = _tc_post(NI, 1)(
      r2(s2_i, 16), r2(cnt_i, 16),
      di2)
  return (h2u, h2i)


# speedup vs baseline: 1.1174x; 1.1174x over previous
import functools

import jax
import jax.numpy as jnp
from jax import lax
from jax.experimental import pallas as pl
from jax.experimental.pallas import tpu as pltpu
from jax.experimental.pallas import tpu_sc as plsc

NU = 50000
NI = 50000
E = 800000
DIN = 128
H = 64
DOUT = 32

NP = 50048
DUMP = 50000
NTILE = 16
TS = NP // NTILE

NB1 = 2

EP2 = 819200
ER2 = EP2 // 128
ROWS2 = ER2 // NTILE
NB2 = 8
NCH2 = ROWS2 // NB2
ROWS_C = ER2 // 2 // NTILE
NBC = 10
NCHC = ROWS_C // NBC

BLK = 2000


def _sc_segsum3(w2, nb, nbuf, er, rows_t):
  mesh = plsc.VectorSubcoreMesh(core_axis_name="c", subcore_axis_name="s")
  out1 = jax.ShapeDtypeStruct((2 * NP, w2), jnp.float32)
  nch = rows_t // nb
  assert nch % nbuf == 0
  buf = lambda: [pltpu.VMEM((nb, 128), jnp.int32),
                 pltpu.VMEM((nb, 128), jnp.int32),
                 pltpu.VMEM((nb, 128, w2), jnp.float32),
                 pltpu.SemaphoreType.DMA,
                 pltpu.SemaphoreType.DMA]
  scratch = []
  for _ in range(nbuf):
    scratch += buf()

  @functools.partial(
      pl.kernel,
      out_type=(out1, out1, out1),
      mesh=mesh,
      compiler_params=pltpu.CompilerParams(use_tc_tiling_on_sc=False),
      scratch_types=scratch + [pltpu.VMEM_SHARED((NP, w2), jnp.float32)],
  )
  def k(*refs):
    (ya, srca, dsta, yb, srcb, dstb, yc, srcc, dstc, zeros_hbm,
     outa, outb, outc) = refs[:13]
    acc = refs[-1]
    bufs = tuple(tuple(refs[13 + 5 * q:13 + 5 * (q + 1)])
                 for q in range(nbuf))
    c = lax.axis_index("c")
    s = lax.axis_index("s")

    def drain_sc(rows, ssem):
      for j in range(nb):
        pltpu.make_async_copy(rows.at[j], acc.at[pl.ds(0, 128)], ssem).wait()

    for y, src2, dst2, out in ((ya, srca, dsta, outa),
                               (yb, srcb, dstb, outb),
                               (yc, srcc, dstc, outc)):
      pltpu.sync_copy(zeros_hbm.at[pl.ds(0, TS)], acc.at[pl.ds(s * TS, TS)])
      plsc.subcore_barrier()
      src_base = c * er + s * rows_t
      dst_base = s * rows_t

      def ring(i2, carry):
        gcps = []
        for b, (srcv, dstv, rows, gsem, ssem) in enumerate(bufs):
          ci = nbuf * i2 + b

          @pl.when(i2 > 0)
          def _():
            drain_sc(rows, ssem)

          pltpu.sync_copy(src2.at[pl.ds(src_base + ci * nb, nb)], srcv)
          pltpu.sync_copy(dst2.at[pl.ds(dst_base + ci * nb, nb)], dstv)
          gcps.append([pltpu.async_copy(y.at[srcv.at[j]], rows.at[j], gsem)
                       for j in range(nb)])
        for b, (srcv, dstv, rows, gsem, ssem) in enumerate(bufs):
          for cp in gcps[b]:
            cp.wait()
          for j in range(nb):
            pltpu.async_copy(rows.at[j], acc.at[dstv.at[j]], ssem, add=True)
        return carry

      lax.fori_loop(0, nch // nbuf, ring, 0)
      for (srcv, dstv, rows, gsem, ssem) in bufs:
        drain_sc(rows, ssem)
      plsc.subcore_barrier()
      pltpu.sync_copy(acc.at[pl.ds(s * TS, TS)],
                      out.at[pl.ds(c * NP + s * TS, TS)])
    return

  return k


def _sc_counts():
  mesh = plsc.VectorSubcoreMesh(core_axis_name="c", subcore_axis_name="s")
  out1 = jax.ShapeDtypeStruct((2 * NP, 16), jnp.float32)

  @functools.partial(
      pl.kernel,
      out_type=(out1, out1, out1),
      mesh=mesh,
      compiler_params=pltpu.CompilerParams(use_tc_tiling_on_sc=False),
      scratch_types=[
          pltpu.VMEM((NBC, 128), jnp.int32),
          pltpu.VMEM((NBC, 128), jnp.int32),
          pltpu.VMEM((128, 16), jnp.float32),
          pltpu.VMEM_SHARED((NP, 16), jnp.float32),
          pltpu.SemaphoreType.DMA,
          pltpu.SemaphoreType.DMA,
      ],
  )
  def k(dsta, dstb, dstc, ones_hbm, zeros_hbm,
        outa, outb, outc, dstv0, dstv1, ones, acc, sem0, sem1):
    c = lax.axis_index("c")
    s = lax.axis_index("s")
    pltpu.sync_copy(ones_hbm, ones)
    bufs = ((dstv0, sem0), (dstv1, sem1))

    def drain(sem):
      for j in range(NBC):
        pltpu.make_async_copy(ones, acc.at[pl.ds(0, 128)], sem).wait()

    for dst2, out in ((dsta, outa), (dstb, outb), (dstc, outc)):
      pltpu.sync_copy(zeros_hbm.at[pl.ds(0, TS)], acc.at[pl.ds(s * TS, TS)])
      plsc.subcore_barrier()
      base = c * (ER2 // 2) + s * ROWS_C

      def pair(i2, carry):
        for b, (dstv, sem) in enumerate(bufs):
          @pl.when(i2 > 0)
          def _():
            drain(sem)

          pltpu.sync_copy(dst2.at[pl.ds(base + (2 * i2 + b) * NBC, NBC)], dstv)
          for j in range(NBC):
            pltpu.async_copy(ones, acc.at[dstv.at[j]], sem, add=True)
        return carry

      lax.fori_loop(0, NCHC // 2, pair, 0)
      for dstv, sem in bufs:
        drain(sem)
      plsc.subcore_barrier()
      pltpu.sync_copy(acc.at[pl.ds(s * TS, TS)],
                      out.at[pl.ds(c * NP + s * TS, TS)])
    return

  return k


_PREC = lax.Precision.DEFAULT


def _full(shape):
  return pl.BlockSpec(shape, lambda i: (0,) * len(shape))


def _dot(a, b):
  return jnp.dot(a, b, preferred_element_type=jnp.float32, precision=_PREC)


def _tc_pre(n, ny):
  grid = n // BLK
  in_specs = [pl.BlockSpec((BLK, DIN), lambda i: (i, 0)),
              _full((DIN, H)), _full((1, H))]
  in_specs += [_full((H, H))] * ny
  in_specs += [_full((H, H)), _full((1, H))]
  out_shape = tuple([jax.ShapeDtypeStruct((2, n, 32), jnp.float32)] * ny
                    + [jax.ShapeDtypeStruct((n, H), jnp.float32)])
  out_specs = tuple([pl.BlockSpec((2, BLK, 32), lambda i: (0, i, 0))] * ny
                    + [pl.BlockSpec((BLK, H), lambda i: (i, 0))])

  def body(*refs):
    x, pwt, pb = refs[0], refs[1], refs[2]
    wls = refs[3:3 + ny]
    wrt, blc = refs[3 + ny], refs[4 + ny]
    youts = refs[5 + ny:5 + 2 * ny]
    duo = refs[5 + 2 * ny]
    h = _dot(x[...], pwt[...]) + pb[...]
    for wl, yo in zip(wls, youts):
      yv = _dot(h, wl[...])
      yo[0] = yv[:, :32]
      yo[1] = yv[:, 32:]
    duo[...] = _dot(h, wrt[...]) + blc[...]

  return pl.pallas_call(body, grid=(grid,), in_specs=in_specs,
                        out_specs=out_specs, out_shape=out_shape)


def _seg_spec(w2):
  return pl.BlockSpec((2, BLK, w2), lambda i: (0, i, 0))


def _agg(p_ref, c_ref):
  inv = 1.0 / jnp.maximum(c_ref[0, :, 0:1] + c_ref[1, :, 0:1], 1.0)
  return jnp.concatenate([p_ref[0], p_ref[1]], axis=1) * inv


def _tc_mid(n, nrel, ny):
  grid = n // BLK
  in_specs = []
  for _ in range(nrel):
    in_specs += [_seg_spec(32), _seg_spec(16)]
  in_specs += [pl.BlockSpec((BLK, H), lambda i: (i, 0))]
  in_specs += [_full((H, DOUT))] * ny
  in_specs += [_full((H, DOUT)), _full((1, DOUT))]
  out_shape = tuple([jax.ShapeDtypeStruct((2, n, 16), jnp.float32)] * ny
                    + [jax.ShapeDtypeStruct((n, DOUT), jnp.float32)])
  out_specs = tuple([pl.BlockSpec((2, BLK, 16), lambda i: (0, i, 0))] * ny
                    + [pl.BlockSpec((BLK, DOUT), lambda i: (i, 0))])
  scale = 1.0 / nrel

  def body(*refs):
    pre = None
    for r in range(nrel):
      a = _agg(refs[2 * r], refs[2 * r + 1])
      pre = a if pre is None else pre + a
    d = refs[2 * nrel]
    wls = refs[2 * nrel + 1:2 * nrel + 1 + ny]
    wrt, blc = refs[2 * nrel + 1 + ny], refs[2 * nrel + 2 + ny]
    youts = refs[2 * nrel + 3 + ny:2 * nrel + 3 + 2 * ny]
    duo = refs[2 * nrel + 3 + 2 * ny]
    h1 = jnp.maximum((pre + d[...]) * scale, 0.0)
    for wl, yo in zip(wls, youts):
      yv = _dot(h1, wl[...])
      yo[0] = yv[:, :16]
      yo[1] = yv[:, 16:]
    duo[...] = _dot(h1, wrt[...]) + blc[...]

  return pl.pallas_call(body, grid=(grid,), in_specs=in_specs,
                        out_specs=out_specs, out_shape=out_shape)


def _tc_post(n, nrel):
  grid = n // BLK
  in_specs = []
  for _ in range(nrel):
    in_specs += [_seg_spec(16), _seg_spec(16)]
  in_specs += [pl.BlockSpec((BLK, DOUT), lambda i: (i, 0))]
  out_shape = jax.ShapeDtypeStruct((n, DOUT), jnp.float32)
  out_specs = pl.BlockSpec((BLK, DOUT), lambda i: (i, 0))
  scale = 1.0 / nrel

  def body(*refs):
    pre = None
    for r in range(nrel):
      a = _agg(refs[2 * r], refs[2 * r + 1])
      pre = a if pre is None else pre + a
    d = refs[2 * nrel]
    refs[2 * nrel + 1][...] = jnp.maximum((pre + d[...]) * scale, 0.0)

  return pl.pallas_call(body, grid=(grid,), in_specs=in_specs,
                        out_specs=out_specs, out_shape=out_shape)


def _prep_edges(ei, n_src):
  src = ei[0].astype(jnp.int32)
  dst = ei[1].astype(jnp.int32)
  p2 = EP2 - E
  src2 = jnp.concatenate([src, jnp.zeros((p2,), jnp.int32)])
  dst2 = jnp.concatenate([dst, jnp.full((p2,), DUMP, jnp.int32)])
  sb = jnp.concatenate([src2, src2 + n_src]).reshape(2 * ER2, 128)
  return sb, dst2.reshape(ER2, 128)


def kernel(x_user, x_item, edge_index_social, edge_index_interacts,
           edge_index_rev_interacts, up_W, up_b, ip_W, ip_b,
           c1s_Wl, c1s_bl, c1s_Wr, c1i_Wl, c1i_bl, c1i_Wr,
           c1r_Wl, c1r_bl, c1r_Wr,
           c2s_Wl, c2s_bl, c2s_Wr, c2i_Wl, c2i_bl, c2i_Wr,
           c2r_Wl, c2r_bl, c2r_Wr):
  sb_s, db_s = _prep_edges(edge_index_social, NU)
  sb_i, db_i = _prep_edges(edge_index_interacts, NU)
  sb_r, db_r = _prep_edges(edge_index_rev_interacts, NI)

  zeros32 = jnp.zeros((TS, 32), jnp.float32)
  zeros16 = jnp.zeros((TS, 16), jnp.float32)
  ones128 = jnp.ones((128, 16), jnp.float32)

  ys1, yi1, du1 = _tc_pre(NU, 2)(
      x_user, up_W.T, up_b.reshape(1, H),
      c1s_Wl.T, c1i_Wl.T,
      (c1s_Wr + c1r_Wr).T, (c1s_bl + c1r_bl).reshape(1, H))
  yr1, di1 = _tc_pre(NI, 1)(
      x_item, ip_W.T, ip_b.reshape(1, H),
      c1r_Wl.T,
      c1i_Wr.T, c1i_bl.reshape(1, H))

  cnt_s, cnt_i, cnt_r = _sc_counts()(db_s, db_i, db_r, ones128, zeros16)
  seg_s, seg_i, seg_r = _sc_segsum3(32, NB1, 2, ER2, ROWS2)(
      ys1.reshape(2 * NU, 32), sb_s, db_s,
      yi1.reshape(2 * NU, 32), sb_i, db_i,
      yr1.reshape(2 * NI, 32), sb_r, db_r,
      zeros32)

  r2 = lambda a, w: a.reshape(2, NP, w)
  ys2, yi2, du2 = _tc_mid(NU, 2, 2)(
      r2(seg_s, 32), r2(cnt_s, 16),
      r2(seg_r, 32), r2(cnt_r, 16),
      du1,
      c2s_Wl.T, c2i_Wl.T,
      (c2s_Wr + c2r_Wr).T, (c2s_bl + c2r_bl).reshape(1, DOUT))
  yr2, di2 = _tc_mid(NI, 1, 1)(
      r2(seg_i, 32), r2(cnt_i, 16),
      di1,
      c2r_Wl.T,
      c2i_Wr.T, c2i_bl.reshape(1, DOUT))

  s2_s, s2_i, s2_r = _sc_segsum3(16, NB2, 2, ER2, ROWS2)(
      ys2.reshape(2 * NU, 16), sb_s, db_s,
      yi2.reshape(2 * NU, 16), sb_i, db_i,
      yr2.reshape(2 * NI, 16), sb_r, db_r,
      zeros16)

  h2u = _tc_post(NU, 2)(
      r2(s2_s, 16), r2(cnt_s, 16),
      r2(s2_r, 16), r2(cnt_r, 16),
      du2)
  h2i = _tc_post(NI, 1)(
      r2(s2_i, 16), r2(cnt_i, 16),
      di2)
  return (h2u, h2i)
